# baseline (speedup 1.0000x reference)
import functools

import jax
import jax.numpy as jnp
from jax import lax
from jax.experimental import pallas as pl
from jax.experimental.pallas import tpu as pltpu
from jax.experimental.pallas import tpu_sc as plsc

NN = 10000
EE = 320000
HH = 8
DD = 16
HD = HH * DD
CLAMP = 5.0

NW = 32
EPW = EE // NW
CB = 80
NCHUNK = EPW // CB
CBA = 40
NCHUNKA = EPW // CBA
NP = 10240
ROWS_PER_SUB = NP // 16
NPR = NP // 8
DRPS = NPR // 16


def _linear_body(x_ref, w_ref, b_ref, o_ref):
    o_ref[...] = (
        jnp.dot(x_ref[...], w_ref[...], preferred_element_type=jnp.float32)
        + b_ref[...]
    )


def _linear(x, w, b, rb):
    m, k = x.shape
    n = w.shape[1]
    return pl.pallas_call(
        _linear_body,
        grid=(m // rb,),
        in_specs=[
            pl.BlockSpec((rb, k), lambda i: (i, 0)),
            pl.BlockSpec((k, n), lambda i: (0, 0)),
            pl.BlockSpec((1, n), lambda i: (0, 0)),
        ],
        out_specs=pl.BlockSpec((rb, n), lambda i: (i, 0)),
        out_shape=jax.ShapeDtypeStruct((m, n), jnp.float32),
    )(x, w, b.reshape(1, -1))


def _edgew_body(c_ref, dst_ref, ewbd_ref, awm16_ref, awm8_ref, awm8t_ref,
                tw_ref, w8_ref, denr_ref, dst8_ref):
    c = c_ref[...]
    t = jnp.dot(c, ewbd_ref[...], preferred_element_type=jnp.float32)
    w16 = jnp.exp(jnp.clip(
        jnp.dot(c, awm16_ref[...], preferred_element_type=jnp.float32),
        -CLAMP, CLAMP))
    tw_ref[...] = w16 * t
    w8_ref[...] = jnp.exp(jnp.clip(
        jnp.dot(c, awm8_ref[...], preferred_element_type=jnp.float32),
        -CLAMP, CLAMP))
    w128 = jnp.exp(jnp.clip(
        jnp.dot(c, awm8t_ref[...], preferred_element_type=jnp.float32),
        -CLAMP, CLAMP))
    lane = lax.broadcasted_iota(jnp.int32, w128.shape, 1)
    slot = lax.rem(dst_ref[...], jnp.int32(8))
    denr_ref[...] = jnp.where((lane >> 4) == slot, w128, 0.0)
    dst8_ref[...] = dst_ref[...] >> 3


def _edge_weights(conn, dst2d, ewbd, awm16, awm8, awm8t, rb):
    return pl.pallas_call(
        _edgew_body,
        grid=(EE // rb,),
        in_specs=[
            pl.BlockSpec((rb, HD), lambda i: (i, 0)),
            pl.BlockSpec((rb, 1), lambda i: (i, 0)),
            pl.BlockSpec((HD, HD), lambda i: (0, 0)),
            pl.BlockSpec((HD, HD), lambda i: (0, 0)),
            pl.BlockSpec((HD, 16), lambda i: (0, 0)),
            pl.BlockSpec((HD, HD), lambda i: (0, 0)),
        ],
        out_specs=[
            pl.BlockSpec((rb, HD), lambda i: (i, 0)),
            pl.BlockSpec((rb, 16), lambda i: (i, 0)),
            pl.BlockSpec((rb, HD), lambda i: (i, 0)),
            pl.BlockSpec((rb, 1), lambda i: (i, 0)),
        ],
        out_shape=[
            jax.ShapeDtypeStruct((EE, HD), jnp.float32),
            jax.ShapeDtypeStruct((EE, 16), jnp.float32),
            jax.ShapeDtypeStruct((EE, HD), jnp.float32),
            jax.ShapeDtypeStruct((EE, 1), jnp.int32),
        ],
    )(conn, dst2d, ewbd, awm16, awm8, awm8t)


def _final_body(agg_ref, den_ref, r8_ref, o_ref):
    agg = agg_ref[0] + agg_ref[1]
    den = den_ref[0] + den_ref[1]
    d16 = jnp.dot(den, r8_ref[...], preferred_element_type=jnp.float32)
    o_ref[...] = agg / (d16 + 1e-16)


def _finalize(agg_p, den_p, r8, rb):
    return pl.pallas_call(
        _final_body,
        grid=(NN // rb,),
        in_specs=[
            pl.BlockSpec((2, rb, HD), lambda i: (0, i, 0)),
            pl.BlockSpec((2, rb, 16), lambda i: (0, i, 0)),
            pl.BlockSpec((16, HD), lambda i: (0, 0)),
        ],
        out_specs=pl.BlockSpec((rb, HD), lambda i: (i, 0)),
        out_shape=jax.ShapeDtypeStruct((NN, HD), jnp.float32),
    )(agg_p, den_p, r8)


_MESH = plsc.VectorSubcoreMesh(core_axis_name="c", subcore_axis_name="s")


@functools.partial(
    pl.kernel,
    mesh=_MESH,
    out_type=jax.ShapeDtypeStruct((EE, HD), jnp.float32),
    scratch_types=[
        pltpu.VMEM((2, CB), jnp.int32),
        pltpu.VMEM((2, CB), jnp.int32),
        pltpu.VMEM((2, CB, HD), jnp.float32),
        pltpu.VMEM((2, CB, HD), jnp.float32),
        pltpu.VMEM((2, CB, HD), jnp.float32),
        pltpu.SemaphoreType.DMA,
        pltpu.SemaphoreType.DMA,
    ],
)
def _conn_sc(nk_hbm, nq_hbm, eq_hbm, src_hbm, dst_hbm, out_hbm,
             srcs, dsts, nks, nqs, cns, g0, g1):
    wid = lax.axis_index("s") * 2 + lax.axis_index("c")

    def issue(i, b, sem):
        base = wid * EPW + i * CB
        pltpu.sync_copy(src_hbm.at[pl.ds(base, CB)], srcs.at[b])
        pltpu.sync_copy(dst_hbm.at[pl.ds(base, CB)], dsts.at[b])
        pltpu.async_copy(nk_hbm.at[srcs.at[b]], nks.at[b], sem)
        pltpu.async_copy(nq_hbm.at[dsts.at[b]], nqs.at[b], sem)
        pltpu.async_copy(eq_hbm.at[pl.ds(base, CB)], cns.at[b], sem)

    def drain(b, sem):
        pltpu.make_async_copy(nk_hbm.at[srcs.at[b]], nks.at[b], sem).wait()
        pltpu.make_async_copy(nq_hbm.at[dsts.at[b]], nqs.at[b], sem).wait()
        pltpu.make_async_copy(eq_hbm.at[pl.ds(0, CB)], cns.at[b], sem).wait()

    def compute(i, b):
        def edge(e, _):
            for h in range(HH):
                sl = (b, e, pl.ds(h * DD, DD))
                cns[sl] = jnp.maximum(nks[sl] + nqs[sl] + cns[sl], 0.0)
            return 0

        lax.fori_loop(0, CB, edge, 0)
        base = wid * EPW + i * CB
        pltpu.sync_copy(cns.at[b], out_hbm.at[pl.ds(base, CB)])

    issue(0, 0, g0)

    def pair(p, _):
        issue(2 * p + 1, 1, g1)
        drain(0, g0)
        compute(2 * p, 0)
        issue(2 * p + 2, 0, g0)
        drain(1, g1)
        compute(2 * p + 1, 1)
        return 0

    lax.fori_loop(0, (NCHUNK - 1) // 2, pair, 0)
    drain(0, g0)
    compute(NCHUNK - 1, 0)


@functools.partial(
    pl.kernel,
    mesh=_MESH,
    out_type=(
        jax.ShapeDtypeStruct((2, NP, HD), jnp.float32),
        jax.ShapeDtypeStruct((2, NPR, HD), jnp.float32),
    ),
    scratch_types=[
        pltpu.VMEM((2, CBA), jnp.int32),
        pltpu.VMEM((2, CBA), jnp.int32),
        pltpu.VMEM((2, CBA), jnp.int32),
        pltpu.VMEM((2, CBA, HD), jnp.float32),
        pltpu.VMEM((2, CBA, HD), jnp.float32),
        pltpu.VMEM((2, CBA, HD), jnp.float32),
        pltpu.VMEM((2, CBA * 16), jnp.float32),
        pltpu.VMEM_SHARED((NP, HD), jnp.float32),
        pltpu.VMEM_SHARED((NPR, HD), jnp.float32),
        pltpu.SemaphoreType.DMA,
        pltpu.SemaphoreType.DMA,
    ],
)
def _agg_sc(tw_hbm, w8f_hbm, denr_hbm, nv_hbm, src_hbm, dst_hbm, dst8_hbm,
            z128_hbm, agg_out, den_out,
            srcs, dsts, dst8s, tws, nvs, denrs, w8fs, agg_sh, den_sh,
            g0, g1):
    cid = lax.axis_index("c")
    sid = lax.axis_index("s")
    wid = sid * 2 + cid
    row0 = pl.multiple_of(sid * ROWS_PER_SUB, ROWS_PER_SUB)
    drow0 = pl.multiple_of(sid * DRPS, DRPS)

    pltpu.sync_copy(z128_hbm, tws.at[0])
    for k in range(ROWS_PER_SUB // CBA):
        pltpu.sync_copy(tws.at[0], agg_sh.at[pl.ds(row0 + k * CBA, CBA)])
    for k in range(DRPS // CBA):
        pltpu.sync_copy(tws.at[0], den_sh.at[pl.ds(drow0 + k * CBA, CBA)])
    plsc.subcore_barrier()

    def issue(i, b, sem):
        base = wid * EPW + i * CBA
        pltpu.sync_copy(src_hbm.at[pl.ds(base, CBA)], srcs.at[b])
        pltpu.sync_copy(dst_hbm.at[pl.ds(base, CBA)], dsts.at[b])
        pltpu.sync_copy(dst8_hbm.at[pl.ds(base, CBA)], dst8s.at[b])
        pltpu.async_copy(nv_hbm.at[srcs.at[b]], nvs.at[b], sem)
        pltpu.async_copy(tw_hbm.at[pl.ds(base, CBA)], tws.at[b], sem)
        pltpu.async_copy(denr_hbm.at[pl.ds(base, CBA)], denrs.at[b], sem)
        pltpu.async_copy(w8f_hbm.at[pl.ds(base * 16, CBA * 16)],
                         w8fs.at[b], sem)

    def drain(b, sem):
        pltpu.make_async_copy(nv_hbm.at[srcs.at[b]], nvs.at[b], sem).wait()
        pltpu.make_async_copy(tw_hbm.at[pl.ds(0, CBA)], tws.at[b], sem).wait()
        pltpu.make_async_copy(denr_hbm.at[pl.ds(0, CBA)], denrs.at[b],
                              sem).wait()
        pltpu.make_async_copy(w8f_hbm.at[pl.ds(0, CBA * 16)], w8fs.at[b],
                              sem).wait()

    def compute(b):
        def edge(e, _):
            wvec = w8fs[b, pl.ds(e * 16, 16)]
            for h in range(HH):
                sl = (b, e, pl.ds(h * DD, DD))
                tws[sl] = tws[sl] + wvec[h] * nvs[sl]
            return 0

        lax.fori_loop(0, CBA, edge, 0)
        pltpu.sync_copy(denrs.at[b], den_sh.at[dst8s.at[b]], add=True)
        pltpu.sync_copy(tws.at[b], agg_sh.at[dsts.at[b]], add=True)

    issue(0, 0, g0)

    def pair(p, _):
        issue(2 * p + 1, 1, g1)
        drain(0, g0)
        compute(0)
        issue(2 * p + 2, 0, g0)
        drain(1, g1)
        compute(1)
        return 0

    lax.fori_loop(0, NCHUNKA // 2 - 1, pair, 0)
    issue(NCHUNKA - 1, 1, g1)
    drain(0, g0)
    compute(0)
    drain(1, g1)
    compute(1)
    plsc.subcore_barrier()

    for k in range(ROWS_PER_SUB // CBA):
        r = row0 + k * CBA
        pltpu.sync_copy(agg_sh.at[pl.ds(r, CBA)], tws.at[0])
        pltpu.sync_copy(tws.at[0], agg_out.at[cid, pl.ds(r, CBA)])
    for k in range(DRPS // CBA):
        r = drow0 + k * CBA
        pltpu.sync_copy(den_sh.at[pl.ds(r, CBA)], denrs.at[0])
        pltpu.sync_copy(denrs.at[0], den_out.at[cid, pl.ds(r, CBA)])


def kernel(x, edge_index, edge_attr, Wq, bq, Wk, bk, Wv, bv, We, be, Aw, Ew):
    src = edge_index[0]
    dst = edge_index[1]

    awm16 = jnp.zeros((HD, HD), jnp.float32)
    ewbd = jnp.zeros((HD, HD), jnp.float32)
    awm8 = jnp.zeros((HD, 16), jnp.float32)
    r8 = jnp.zeros((16, HD), jnp.float32)
    for h in range(HH):
        blk = Aw[:, h, 0:1] * jnp.ones((1, DD), jnp.float32)
        awm16 = awm16.at[h * DD:(h + 1) * DD, h * DD:(h + 1) * DD].set(blk)
        ewbd = ewbd.at[h * DD:(h + 1) * DD, h * DD:(h + 1) * DD].set(Ew[:, h, :])
        awm8 = awm8.at[h * DD:(h + 1) * DD, h].set(Aw[:, h, 0])
        awm8 = awm8.at[h * DD:(h + 1) * DD, h + 8].set(Aw[:, h, 0])
        r8 = r8.at[h, h * DD:(h + 1) * DD].set(1.0)

    wqkv = jnp.concatenate([Wq, Wk, Wv], axis=1)
    bqkv = jnp.concatenate([bq, bk, bv], axis=0)

    nqkv = _linear(x, wqkv, bqkv, rb=2000)
    nq = nqkv[:, :HD]
    nk = nqkv[:, HD:2 * HD]
    nv = nqkv[:, 2 * HD:]
    eq = _linear(edge_attr, We, be, rb=2000)

    conn = _conn_sc(nk, nq, eq, src, dst)

    awm8t = jnp.tile(awm8, (1, 8))
    tw, w8, denr, dst8 = _edge_weights(conn, dst.reshape(-1, 1), ewbd,
                                       awm16, awm8, awm8t, rb=2000)

    z128 = jnp.zeros((CBA, HD), jnp.float32)
    agg_p, denr_p = _agg_sc(tw, w8.reshape(-1), denr, nv, src, dst,
                            dst8.reshape(-1), z128)
    den_p = denr_p.reshape(2, NP, 16)

    n_out = _finalize(agg_p[:, :NN], den_p[:, :NN], r8, rb=2000)
    return (n_out, conn)


# speedup vs baseline: 1.1148x; 1.1148x over previous
import functools

import jax
import jax.numpy as jnp
from jax import lax
from jax.experimental import pallas as pl
from jax.experimental.pallas import tpu as pltpu
from jax.experimental.pallas import tpu_sc as plsc

NN = 10000
EE = 320000
HH = 8
DD = 16
HD = HH * DD
CLAMP = 5.0

NW = 32
EPW = EE // NW
CB = 80
NCHUNK = EPW // CB
CBA = 80
NCHUNKA = EPW // CBA
NP = 10240
ROWS_PER_SUB = NP // 16
NPR = NP // 16
DRPS = NPR // 16


def _linear_body(x_ref, w_ref, b_ref, o_ref):
    o_ref[...] = (
        jnp.dot(x_ref[...], w_ref[...], preferred_element_type=jnp.float32)
        + b_ref[...]
    )


def _linear(x, w, b, rb):
    m, k = x.shape
    n = w.shape[1]
    return pl.pallas_call(
        _linear_body,
        grid=(m // rb,),
        in_specs=[
            pl.BlockSpec((rb, k), lambda i: (i, 0)),
            pl.BlockSpec((k, n), lambda i: (0, 0)),
            pl.BlockSpec((1, n), lambda i: (0, 0)),
        ],
        out_specs=pl.BlockSpec((rb, n), lambda i: (i, 0)),
        out_shape=jax.ShapeDtypeStruct((m, n), jnp.float32),
    )(x, w, b.reshape(1, -1))


def _edgew_body(c_ref, dst_ref, ewbd_ref, awm16_ref, awm8_ref, awm8t_ref,
                tw_ref, w8_ref, denr_ref, dst8_ref):
    c = c_ref[...]
    t = jnp.dot(c, ewbd_ref[...], preferred_element_type=jnp.float32)
    w16 = jnp.exp(jnp.clip(
        jnp.dot(c, awm16_ref[...], preferred_element_type=jnp.float32),
        -CLAMP, CLAMP))
    tw_ref[...] = w16 * t
    w8_ref[...] = jnp.exp(jnp.clip(
        jnp.dot(c, awm8_ref[...], preferred_element_type=jnp.float32),
        -CLAMP, CLAMP))
    w128 = jnp.exp(jnp.clip(
        jnp.dot(c, awm8t_ref[...], preferred_element_type=jnp.float32),
        -CLAMP, CLAMP))
    lane = lax.broadcasted_iota(jnp.int32, w128.shape, 1)
    slot = lax.rem(dst_ref[...], jnp.int32(16))
    denr_ref[...] = jnp.where((lane >> 3) == slot, w128, 0.0)
    dst8_ref[...] = dst_ref[...] >> 4


def _edge_weights(conn, dst2d, ewbd, awm16, awm8, awm8t, rb):
    return pl.pallas_call(
        _edgew_body,
        grid=(EE // rb,),
        in_specs=[
            pl.BlockSpec((rb, HD), lambda i: (i, 0)),
            pl.BlockSpec((rb, 1), lambda i: (i, 0)),
            pl.BlockSpec((HD, HD), lambda i: (0, 0)),
            pl.BlockSpec((HD, HD), lambda i: (0, 0)),
            pl.BlockSpec((HD, 16), lambda i: (0, 0)),
            pl.BlockSpec((HD, HD), lambda i: (0, 0)),
        ],
        out_specs=[
            pl.BlockSpec((rb, HD), lambda i: (i, 0)),
            pl.BlockSpec((rb, 16), lambda i: (i, 0)),
            pl.BlockSpec((rb, HD), lambda i: (i, 0)),
            pl.BlockSpec((rb, 1), lambda i: (i, 0)),
        ],
        out_shape=[
            jax.ShapeDtypeStruct((EE, HD), jnp.float32),
            jax.ShapeDtypeStruct((EE, 16), jnp.float32),
            jax.ShapeDtypeStruct((EE, HD), jnp.float32),
            jax.ShapeDtypeStruct((EE, 1), jnp.int32),
        ],
    )(conn, dst2d, ewbd, awm16, awm8, awm8t)


def _final_body(agg_ref, den_ref, r8_ref, o_ref):
    agg = agg_ref[0] + agg_ref[1]
    den = den_ref[0] + den_ref[1]
    d16 = jnp.dot(den, r8_ref[...], preferred_element_type=jnp.float32)
    o_ref[...] = agg / (d16 + 1e-16)


def _finalize(agg_p, den_p, r8, rb):
    return pl.pallas_call(
        _final_body,
        grid=(NN // rb,),
        in_specs=[
            pl.BlockSpec((2, rb, HD), lambda i: (0, i, 0)),
            pl.BlockSpec((2, rb, 8), lambda i: (0, i, 0)),
            pl.BlockSpec((8, HD), lambda i: (0, 0)),
        ],
        out_specs=pl.BlockSpec((rb, HD), lambda i: (i, 0)),
        out_shape=jax.ShapeDtypeStruct((NN, HD), jnp.float32),
    )(agg_p, den_p, r8)


_MESH = plsc.VectorSubcoreMesh(core_axis_name="c", subcore_axis_name="s")


@functools.partial(
    pl.kernel,
    mesh=_MESH,
    out_type=jax.ShapeDtypeStruct((EE, HD), jnp.float32),
    scratch_types=[
        pltpu.VMEM((2, CB), jnp.int32),
        pltpu.VMEM((2, CB), jnp.int32),
        pltpu.VMEM((2, CB, HD), jnp.float32),
        pltpu.VMEM((2, CB, HD), jnp.float32),
        pltpu.VMEM((2, CB, HD), jnp.float32),
        pltpu.SemaphoreType.DMA,
        pltpu.SemaphoreType.DMA,
    ],
)
def _conn_sc(nk_hbm, nq_hbm, eq_hbm, src_hbm, dst_hbm, out_hbm,
             srcs, dsts, nks, nqs, cns, g0, g1):
    wid = lax.axis_index("s") * 2 + lax.axis_index("c")

    def issue(i, b, sem):
        base = wid * EPW + i * CB
        pltpu.sync_copy(src_hbm.at[pl.ds(base, CB)], srcs.at[b])
        pltpu.sync_copy(dst_hbm.at[pl.ds(base, CB)], dsts.at[b])
        pltpu.async_copy(nk_hbm.at[srcs.at[b]], nks.at[b], sem)
        pltpu.async_copy(nq_hbm.at[dsts.at[b]], nqs.at[b], sem)
        pltpu.async_copy(eq_hbm.at[pl.ds(base, CB)], cns.at[b], sem)

    def drain(b, sem):
        pltpu.make_async_copy(nk_hbm.at[srcs.at[b]], nks.at[b], sem).wait()
        pltpu.make_async_copy(nq_hbm.at[dsts.at[b]], nqs.at[b], sem).wait()
        pltpu.make_async_copy(eq_hbm.at[pl.ds(0, CB)], cns.at[b], sem).wait()

    def compute(i, b):
        def edge(e, _):
            for h in range(HH):
                sl = (b, e, pl.ds(h * DD, DD))
                cns[sl] = jnp.maximum(nks[sl] + nqs[sl] + cns[sl], 0.0)
            return 0

        lax.fori_loop(0, CB, edge, 0)
        base = wid * EPW + i * CB
        pltpu.sync_copy(cns.at[b], out_hbm.at[pl.ds(base, CB)])

    issue(0, 0, g0)

    def pair(p, _):
        issue(2 * p + 1, 1, g1)
        drain(0, g0)
        compute(2 * p, 0)
        issue(2 * p + 2, 0, g0)
        drain(1, g1)
        compute(2 * p + 1, 1)
        return 0

    lax.fori_loop(0, (NCHUNK - 1) // 2, pair, 0)
    drain(0, g0)
    compute(NCHUNK - 1, 0)


@functools.partial(
    pl.kernel,
    mesh=_MESH,
    out_type=(
        jax.ShapeDtypeStruct((2, NP, HD), jnp.float32),
        jax.ShapeDtypeStruct((2, NPR, HD), jnp.float32),
    ),
    scratch_types=[
        pltpu.VMEM((2, CBA), jnp.int32),
        pltpu.VMEM((CBA,), jnp.int32),
        pltpu.VMEM((CBA,), jnp.int32),
        pltpu.VMEM((CBA, HD), jnp.float32),
        pltpu.VMEM((2, CBA, HD), jnp.float32),
        pltpu.VMEM((CBA, HD), jnp.float32),
        pltpu.VMEM((CBA * 16,), jnp.float32),
        pltpu.VMEM_SHARED((NP, HD), jnp.float32),
        pltpu.VMEM_SHARED((NPR, HD), jnp.float32),
        pltpu.SemaphoreType.DMA,
        pltpu.SemaphoreType.DMA,
    ],
)
def _agg_sc(tw_hbm, w8f_hbm, denr_hbm, nv_hbm, src_hbm, dst_hbm, dst8_hbm,
            z128_hbm, agg_out, den_out,
            srcs, dst_v, dst8_v, tw_v, nvs, denr_v, w8f_v, agg_sh, den_sh,
            g0, g1):
    cid = lax.axis_index("c")
    sid = lax.axis_index("s")
    wid = sid * 2 + cid
    row0 = pl.multiple_of(sid * ROWS_PER_SUB, ROWS_PER_SUB)
    drow0 = pl.multiple_of(sid * DRPS, DRPS)

    pltpu.sync_copy(z128_hbm, tw_v)
    for k in range(ROWS_PER_SUB // CBA):
        pltpu.sync_copy(tw_v, agg_sh.at[pl.ds(row0 + k * CBA, CBA)])
    pltpu.sync_copy(tw_v.at[pl.ds(0, DRPS)], den_sh.at[pl.ds(drow0, DRPS)])
    plsc.subcore_barrier()

    def prefetch(i, b, sem):
        base = wid * EPW + i * CBA
        pltpu.sync_copy(src_hbm.at[pl.ds(base, CBA)], srcs.at[b])
        pltpu.async_copy(nv_hbm.at[srcs.at[b]], nvs.at[b], sem)

    def compute(i, b, sem):
        base = wid * EPW + i * CBA
        pltpu.sync_copy(dst_hbm.at[pl.ds(base, CBA)], dst_v)
        pltpu.sync_copy(dst8_hbm.at[pl.ds(base, CBA)], dst8_v)
        pltpu.sync_copy(tw_hbm.at[pl.ds(base, CBA)], tw_v)
        pltpu.sync_copy(denr_hbm.at[pl.ds(base, CBA)], denr_v)
        pltpu.sync_copy(w8f_hbm.at[pl.ds(base * 16, CBA * 16)], w8f_v)
        pltpu.make_async_copy(nv_hbm.at[srcs.at[b]], nvs.at[b], sem).wait()

        def edge(e, _):
            wvec = w8f_v[pl.ds(e * 16, 16)]
            for h in range(HH):
                sl = (e, pl.ds(h * DD, DD))
                tw_v[sl] = tw_v[sl] + wvec[h] * nvs[(b,) + sl]
            return 0

        lax.fori_loop(0, CBA, edge, 0)
        pltpu.sync_copy(denr_v, den_sh.at[dst8_v], add=True)
        pltpu.sync_copy(tw_v, agg_sh.at[dst_v], add=True)

    prefetch(0, 0, g0)

    def pair(p, _):
        prefetch(2 * p + 1, 1, g1)
        compute(2 * p, 0, g0)
        prefetch(2 * p + 2, 0, g0)
        compute(2 * p + 1, 1, g1)
        return 0

    lax.fori_loop(0, (NCHUNKA - 1) // 2, pair, 0)
    compute(NCHUNKA - 1, 0, g0)
    plsc.subcore_barrier()

    for k in range(ROWS_PER_SUB // CBA):
        r = row0 + k * CBA
        pltpu.sync_copy(agg_sh.at[pl.ds(r, CBA)], tw_v)
        pltpu.sync_copy(tw_v, agg_out.at[cid, pl.ds(r, CBA)])
    pltpu.sync_copy(den_sh.at[pl.ds(drow0, DRPS)], denr_v.at[pl.ds(0, DRPS)])
    pltpu.sync_copy(denr_v.at[pl.ds(0, DRPS)], den_out.at[cid, pl.ds(drow0, DRPS)])


def kernel(x, edge_index, edge_attr, Wq, bq, Wk, bk, Wv, bv, We, be, Aw, Ew):
    src = edge_index[0]
    dst = edge_index[1]

    awm16 = jnp.zeros((HD, HD), jnp.float32)
    ewbd = jnp.zeros((HD, HD), jnp.float32)
    awm8 = jnp.zeros((HD, 16), jnp.float32)
    r8 = jnp.zeros((8, HD), jnp.float32)
    for h in range(HH):
        blk = Aw[:, h, 0:1] * jnp.ones((1, DD), jnp.float32)
        awm16 = awm16.at[h * DD:(h + 1) * DD, h * DD:(h + 1) * DD].set(blk)
        ewbd = ewbd.at[h * DD:(h + 1) * DD, h * DD:(h + 1) * DD].set(Ew[:, h, :])
        awm8 = awm8.at[h * DD:(h + 1) * DD, h].set(Aw[:, h, 0])
        awm8 = awm8.at[h * DD:(h + 1) * DD, h + 8].set(Aw[:, h, 0])
        r8 = r8.at[h, h * DD:(h + 1) * DD].set(1.0)

    wqkv = jnp.concatenate([Wq, Wk, Wv], axis=1)
    bqkv = jnp.concatenate([bq, bk, bv], axis=0)

    nqkv = _linear(x, wqkv, bqkv, rb=2000)
    nq = nqkv[:, :HD]
    nk = nqkv[:, HD:2 * HD]
    nv = nqkv[:, 2 * HD:]
    eq = _linear(edge_attr, We, be, rb=2000)

    conn = _conn_sc(nk, nq, eq, src, dst)

    awm8t = jnp.tile(awm8[:, :8], (1, 16))
    tw, w8, denr, dst8 = _edge_weights(conn, dst.reshape(-1, 1), ewbd,
                                       awm16, awm8, awm8t, rb=2000)

    z128 = jnp.zeros((CBA, HD), jnp.float32)
    agg_p, denr_p = _agg_sc(tw, w8.reshape(-1), denr, nv, src, dst,
                            dst8.reshape(-1), z128)
    den_p = denr_p.reshape(2, NP, 8)

    n_out = _finalize(agg_p[:, :NN], den_p[:, :NN], r8, rb=2000)
    return (n_out, conn)
